# baseline (speedup 1.0000x reference)
import functools

import jax
import jax.numpy as jnp
from jax import lax
from jax.experimental import pallas as pl
from jax.experimental.pallas import tpu as pltpu
from jax.experimental.pallas import tpu_sc as plsc

N = 10000
E = 320000
D = 128

NC = 2
NS = 16
NW = NC * NS
BLK = 80
E_PAD = 327680
WBLK = E_PAD // (NW * BLK)
CH = 32
NCHUNK = WBLK // CH
N_PAD = N + 8


def _sc_aggregate(values, zeros, src_w, dst_w):
    mesh = plsc.VectorSubcoreMesh(core_axis_name="c", subcore_axis_name="s")

    @functools.partial(
        pl.kernel,
        out_type=jax.ShapeDtypeStruct((NC, N, D), jnp.float32),
        mesh=mesh,
        scratch_types=[
            pltpu.VMEM((CH, BLK), jnp.int32),
            pltpu.VMEM((CH, BLK), jnp.int32),
            pltpu.VMEM((BLK, D), jnp.float32),
            pltpu.VMEM((BLK, D), jnp.float32),
            pltpu.VMEM_SHARED((N, D), jnp.float32),
            pltpu.SemaphoreType.DMA,
            pltpu.SemaphoreType.DMA,
        ],
    )
    def agg_kernel(x_hbm, z_hbm, src_hbm, dst_hbm, out_hbm,
                   src_v, dst_v, rows0, rows1, acc_sh, sem0, sem1):
        cid = lax.axis_index("c")
        sid = lax.axis_index("s")
        wid = sid * NC + cid

        @pl.when(sid == 0)
        def _():
            pltpu.sync_copy(z_hbm, acc_sh)

        plsc.subcore_barrier()

        wait0 = pltpu.make_async_copy(x_hbm.at[src_v.at[0]], rows0, sem0)
        wait1 = pltpu.make_async_copy(x_hbm.at[src_v.at[0]], rows1, sem1)

        @pl.loop(0, NCHUNK)
        def _(c):
            pltpu.sync_copy(src_hbm.at[wid, pl.ds(c * CH, CH)], src_v)
            pltpu.sync_copy(dst_hbm.at[wid, pl.ds(c * CH, CH)], dst_v)
            pltpu.async_copy(x_hbm.at[src_v.at[0]], rows0, sem0)

            @pl.loop(0, CH // 2 - 1)
            def _(i):
                b = 2 * i
                wait0.wait()
                pltpu.async_copy(x_hbm.at[src_v.at[b + 1]], rows1, sem1)
                pltpu.sync_copy(rows0, acc_sh.at[dst_v.at[b]], add=True)
                wait1.wait()
                pltpu.async_copy(x_hbm.at[src_v.at[b + 2]], rows0, sem0)
                pltpu.sync_copy(rows1, acc_sh.at[dst_v.at[b + 1]], add=True)

            wait0.wait()
            pltpu.async_copy(x_hbm.at[src_v.at[CH - 1]], rows1, sem1)
            pltpu.sync_copy(rows0, acc_sh.at[dst_v.at[CH - 2]], add=True)
            wait1.wait()
            pltpu.sync_copy(rows1, acc_sh.at[dst_v.at[CH - 1]], add=True)

        plsc.subcore_barrier()

        @pl.when(sid == 0)
        def _():
            pltpu.sync_copy(acc_sh, out_hbm.at[cid])

    return agg_kernel(values, zeros, src_w, dst_w)


ROW_BLK = 1000


def _mlp1_body(x_ref, p_ref, w1_ref, b1_ref, w2_ref, b2_ref, o_ref):
    h = x_ref[...] + p_ref[0] + p_ref[1]
    a = lax.dot_general(h, w1_ref[...], (((1,), (0,)), ((), ())),
                        precision=lax.Precision.HIGHEST,
                        preferred_element_type=jnp.float32)
    a = jnp.maximum(a + b1_ref[...], 0.0)
    hh = lax.dot_general(a, w2_ref[...], (((1,), (0,)), ((), ())),
                         precision=lax.Precision.HIGHEST,
                         preferred_element_type=jnp.float32)
    hh = hh + b2_ref[...]
    o_ref[...] = jnp.where(hh > 0, hh, jnp.exp(hh) - 1.0)


def _mlp2_body(h_ref, q_ref, w3_ref, b3_ref, o_ref):
    h2 = h_ref[...] + q_ref[0] + q_ref[1]
    a = lax.dot_general(h2, w3_ref[...], (((1,), (0,)), ((), ())),
                        precision=lax.Precision.HIGHEST,
                        preferred_element_type=jnp.float32)
    o_ref[...] = jnp.maximum(a + b3_ref[...], 0.0)


def _row_spec():
    return pl.BlockSpec((ROW_BLK, D), lambda i: (i, 0))


def _pair_spec():
    return pl.BlockSpec((NC, ROW_BLK, D), lambda i: (0, i, 0))


def _full_spec(shape):
    return pl.BlockSpec(shape, lambda i: tuple(0 for _ in shape))


def _mlp1(x, p, W1, b1, W2, b2):
    return pl.pallas_call(
        _mlp1_body,
        grid=(N // ROW_BLK,),
        in_specs=[_row_spec(), _pair_spec(),
                  _full_spec((D, D)), _full_spec((1, D)),
                  _full_spec((D, D)), _full_spec((1, D))],
        out_specs=_row_spec(),
        out_shape=jax.ShapeDtypeStruct((N, D), jnp.float32),
    )(x, p, W1, b1.reshape(1, D), W2, b2.reshape(1, D))


def _mlp2(h, q, W3, b3):
    return pl.pallas_call(
        _mlp2_body,
        grid=(N // ROW_BLK,),
        in_specs=[_row_spec(), _pair_spec(),
                  _full_spec((D, D)), _full_spec((1, D))],
        out_specs=_row_spec(),
        out_shape=jax.ShapeDtypeStruct((N, D), jnp.float32),
    )(h, q, W3, b3.reshape(1, D))


def kernel(x, edge_index, W1, b1, W2, b2, W3, b3):
    pad = E_PAD - E
    src = jnp.concatenate(
        [edge_index[0].astype(jnp.int32), jnp.full((pad,), N, jnp.int32)]
    ).reshape(NW, WBLK, BLK)
    dst = jnp.concatenate(
        [edge_index[1].astype(jnp.int32),
         (jnp.arange(pad, dtype=jnp.int32) * 13) % N]
    ).reshape(NW, WBLK, BLK)
    zeros = jnp.zeros((N, D), jnp.float32)
    zrows = jnp.zeros((N_PAD - N, D), jnp.float32)

    p = _sc_aggregate(jnp.concatenate([x, zrows]), zeros, src, dst)
    h = _mlp1(x, p, W1, b1, W2, b2)
    q = _sc_aggregate(jnp.concatenate([h, zrows]), zeros, src, dst)
    return _mlp2(h, q, W3, b3)


# speedup vs baseline: 2.8335x; 2.8335x over previous
import functools

import jax
import jax.numpy as jnp
from jax import lax
from jax.experimental import pallas as pl
from jax.experimental.pallas import tpu as pltpu
from jax.experimental.pallas import tpu_sc as plsc

N = 10000
E = 320000
D = 128

NC = 2
NS = 16
NW = NC * NS
BLK = 80
WBLK = E // (NW * BLK)
CH = 25
NCHUNK = WBLK // CH


def _sc_aggregate(values, zeros, src_w, dst_w):
    mesh = plsc.VectorSubcoreMesh(core_axis_name="c", subcore_axis_name="s")

    @functools.partial(
        pl.kernel,
        out_type=jax.ShapeDtypeStruct((NC, N, D), jnp.float32),
        mesh=mesh,
        scratch_types=[
            pltpu.VMEM((CH, BLK), jnp.int32),
            pltpu.VMEM((CH, BLK), jnp.int32),
            pltpu.VMEM((BLK, D), jnp.float32),
            pltpu.VMEM((BLK, D), jnp.float32),
            pltpu.VMEM_SHARED((N, D), jnp.float32),
            pltpu.SemaphoreType.DMA,
            pltpu.SemaphoreType.DMA,
        ],
    )
    def agg_kernel(x_hbm, z_hbm, src_hbm, dst_hbm, out_hbm,
                   src_v, dst_v, rows0, rows1, acc_sh, sem0, sem1):
        cid = lax.axis_index("c")
        sid = lax.axis_index("s")
        wid = sid * NC + cid

        @pl.when(sid == 0)
        def _():
            pltpu.sync_copy(z_hbm, acc_sh)

        plsc.subcore_barrier()

        wait0 = pltpu.make_async_copy(x_hbm.at[src_v.at[0]], rows0, sem0)
        wait1 = pltpu.make_async_copy(x_hbm.at[src_v.at[0]], rows1, sem1)

        @pl.loop(0, NCHUNK)
        def _(c):
            pltpu.sync_copy(src_hbm.at[wid, c], src_v)
            pltpu.sync_copy(dst_hbm.at[wid, c], dst_v)
            pltpu.async_copy(x_hbm.at[src_v.at[0]], rows0, sem0)

            @pl.loop(0, (CH - 1) // 2)
            def _(i):
                b = 2 * i
                wait0.wait()
                pltpu.async_copy(x_hbm.at[src_v.at[b + 1]], rows1, sem1)
                pltpu.sync_copy(rows0, acc_sh.at[dst_v.at[b]], add=True)
                wait1.wait()
                pltpu.async_copy(x_hbm.at[src_v.at[b + 2]], rows0, sem0)
                pltpu.sync_copy(rows1, acc_sh.at[dst_v.at[b + 1]], add=True)

            wait0.wait()
            pltpu.sync_copy(rows0, acc_sh.at[dst_v.at[CH - 1]], add=True)

        plsc.subcore_barrier()

        @pl.when(sid == 0)
        def _():
            pltpu.sync_copy(acc_sh, out_hbm.at[cid])

    return agg_kernel(values, zeros, src_w, dst_w)


ROW_BLK = 1000


def _mlp1_body(x_ref, p_ref, w1_ref, b1_ref, w2_ref, b2_ref, o_ref):
    h = x_ref[...] + p_ref[0] + p_ref[1]
    a = lax.dot_general(h, w1_ref[...], (((1,), (0,)), ((), ())),
                        precision=lax.Precision.HIGHEST,
                        preferred_element_type=jnp.float32)
    a = jnp.maximum(a + b1_ref[...], 0.0)
    hh = lax.dot_general(a, w2_ref[...], (((1,), (0,)), ((), ())),
                         precision=lax.Precision.HIGHEST,
                         preferred_element_type=jnp.float32)
    hh = hh + b2_ref[...]
    o_ref[...] = jnp.where(hh > 0, hh, jnp.exp(hh) - 1.0)


def _mlp2_body(h_ref, q_ref, w3_ref, b3_ref, o_ref):
    h2 = h_ref[...] + q_ref[0] + q_ref[1]
    a = lax.dot_general(h2, w3_ref[...], (((1,), (0,)), ((), ())),
                        precision=lax.Precision.HIGHEST,
                        preferred_element_type=jnp.float32)
    o_ref[...] = jnp.maximum(a + b3_ref[...], 0.0)


def _row_spec():
    return pl.BlockSpec((ROW_BLK, D), lambda i: (i, 0))


def _pair_spec():
    return pl.BlockSpec((NC, ROW_BLK, D), lambda i: (0, i, 0))


def _full_spec(shape):
    return pl.BlockSpec(shape, lambda i: tuple(0 for _ in shape))


def _mlp1(x, p, W1, b1, W2, b2):
    return pl.pallas_call(
        _mlp1_body,
        grid=(N // ROW_BLK,),
        in_specs=[_row_spec(), _pair_spec(),
                  _full_spec((D, D)), _full_spec((1, D)),
                  _full_spec((D, D)), _full_spec((1, D))],
        out_specs=_row_spec(),
        out_shape=jax.ShapeDtypeStruct((N, D), jnp.float32),
    )(x, p, W1, b1.reshape(1, D), W2, b2.reshape(1, D))


def _mlp2(h, q, W3, b3):
    return pl.pallas_call(
        _mlp2_body,
        grid=(N // ROW_BLK,),
        in_specs=[_row_spec(), _pair_spec(),
                  _full_spec((D, D)), _full_spec((1, D))],
        out_specs=_row_spec(),
        out_shape=jax.ShapeDtypeStruct((N, D), jnp.float32),
    )(h, q, W3, b3.reshape(1, D))


def kernel(x, edge_index, W1, b1, W2, b2, W3, b3):
    src = edge_index[0].astype(jnp.int32).reshape(NW, NCHUNK, CH, BLK)
    dst = edge_index[1].astype(jnp.int32).reshape(NW, NCHUNK, CH, BLK)
    zeros = jnp.zeros((N, D), jnp.float32)

    p = _sc_aggregate(x, zeros, src, dst)
    h = _mlp1(x, p, W1, b1, W2, b2)
    q = _sc_aggregate(h, zeros, src, dst)
    return _mlp2(h, q, W3, b3)


# speedup vs baseline: 2.9161x; 1.0291x over previous
import functools

import jax
import jax.numpy as jnp
from jax import lax
from jax.experimental import pallas as pl
from jax.experimental.pallas import tpu as pltpu
from jax.experimental.pallas import tpu_sc as plsc

N = 10000
E = 320000
D = 128

NC = 2
NS = 16
NW = NC * NS
BLK = 80
WBLK = E // (NW * BLK)
CH = 25
NCHUNK = WBLK // CH


def _sc_aggregate(values, zeros, edges):
    mesh = plsc.VectorSubcoreMesh(core_axis_name="c", subcore_axis_name="s")

    @functools.partial(
        pl.kernel,
        out_type=jax.ShapeDtypeStruct((NC, N, D), jnp.float32),
        mesh=mesh,
        scratch_types=[
            pltpu.VMEM((CH, BLK), jnp.int32),
            pltpu.VMEM((CH, BLK), jnp.int32),
            pltpu.VMEM((BLK, D), jnp.float32),
            pltpu.VMEM((BLK, D), jnp.float32),
            pltpu.VMEM_SHARED((N, D), jnp.float32),
            pltpu.SemaphoreType.DMA,
            pltpu.SemaphoreType.DMA,
        ],
    )
    def agg_kernel(x_hbm, z_hbm, e_hbm, out_hbm,
                   src_v, dst_v, rows0, rows1, acc_sh, sem0, sem1):
        cid = lax.axis_index("c")
        sid = lax.axis_index("s")
        wid = sid * NC + cid

        @pl.when(sid == 0)
        def _():
            pltpu.sync_copy(z_hbm, acc_sh)

        plsc.subcore_barrier()

        wait0 = pltpu.make_async_copy(x_hbm.at[src_v.at[0]], rows0, sem0)
        wait1 = pltpu.make_async_copy(x_hbm.at[src_v.at[0]], rows1, sem1)

        @pl.loop(0, NCHUNK)
        def _(c):
            pltpu.sync_copy(e_hbm.at[0, wid, c], src_v)
            pltpu.sync_copy(e_hbm.at[1, wid, c], dst_v)
            pltpu.async_copy(x_hbm.at[src_v.at[0]], rows0, sem0)

            @pl.loop(0, (CH - 1) // 2)
            def _(i):
                b = 2 * i
                wait0.wait()
                pltpu.async_copy(x_hbm.at[src_v.at[b + 1]], rows1, sem1)
                pltpu.sync_copy(rows0, acc_sh.at[dst_v.at[b]], add=True)
                wait1.wait()
                pltpu.async_copy(x_hbm.at[src_v.at[b + 2]], rows0, sem0)
                pltpu.sync_copy(rows1, acc_sh.at[dst_v.at[b + 1]], add=True)

            wait0.wait()
            pltpu.sync_copy(rows0, acc_sh.at[dst_v.at[CH - 1]], add=True)

        plsc.subcore_barrier()

        @pl.when(sid == 0)
        def _():
            pltpu.sync_copy(acc_sh, out_hbm.at[cid])

    return agg_kernel(values, zeros, edges)


ROW_BLK = 1000


def _mlp1_body(x_ref, p_ref, w1_ref, b1_ref, w2_ref, b2_ref, o_ref):
    h = x_ref[...] + p_ref[0] + p_ref[1]
    a = lax.dot_general(h, w1_ref[...], (((1,), (0,)), ((), ())),
                        precision=lax.Precision.HIGHEST,
                        preferred_element_type=jnp.float32)
    a = jnp.maximum(a + b1_ref[...], 0.0)
    hh = lax.dot_general(a, w2_ref[...], (((1,), (0,)), ((), ())),
                         precision=lax.Precision.HIGHEST,
                         preferred_element_type=jnp.float32)
    hh = hh + b2_ref[...]
    o_ref[...] = jnp.where(hh > 0, hh, jnp.exp(hh) - 1.0)


def _mlp2_body(h_ref, q_ref, w3_ref, b3_ref, o_ref):
    h2 = h_ref[...] + q_ref[0] + q_ref[1]
    a = lax.dot_general(h2, w3_ref[...], (((1,), (0,)), ((), ())),
                        precision=lax.Precision.HIGHEST,
                        preferred_element_type=jnp.float32)
    o_ref[...] = jnp.maximum(a + b3_ref[...], 0.0)


def _row_spec():
    return pl.BlockSpec((ROW_BLK, D), lambda i: (i, 0))


def _pair_spec():
    return pl.BlockSpec((NC, ROW_BLK, D), lambda i: (0, i, 0))


def _full_spec(shape):
    return pl.BlockSpec(shape, lambda i: tuple(0 for _ in shape))


def _mlp1(x, p, W1, b1, W2, b2):
    return pl.pallas_call(
        _mlp1_body,
        grid=(N // ROW_BLK,),
        in_specs=[_row_spec(), _pair_spec(),
                  _full_spec((D, D)), _full_spec((1, D)),
                  _full_spec((D, D)), _full_spec((1, D))],
        out_specs=_row_spec(),
        out_shape=jax.ShapeDtypeStruct((N, D), jnp.float32),
        compiler_params=pltpu.CompilerParams(
            dimension_semantics=("parallel",)),
    )(x, p, W1, b1.reshape(1, D), W2, b2.reshape(1, D))


def _mlp2(h, q, W3, b3):
    return pl.pallas_call(
        _mlp2_body,
        grid=(N // ROW_BLK,),
        in_specs=[_row_spec(), _pair_spec(),
                  _full_spec((D, D)), _full_spec((1, D))],
        out_specs=_row_spec(),
        out_shape=jax.ShapeDtypeStruct((N, D), jnp.float32),
        compiler_params=pltpu.CompilerParams(
            dimension_semantics=("parallel",)),
    )(h, q, W3, b3.reshape(1, D))


def kernel(x, edge_index, W1, b1, W2, b2, W3, b3):
    edges = edge_index.astype(jnp.int32).reshape(2, NW, NCHUNK, CH, BLK)
    zeros = jnp.zeros((N, D), jnp.float32)

    p = _sc_aggregate(x, zeros, edges)
    h = _mlp1(x, p, W1, b1, W2, b2)
    q = _sc_aggregate(h, zeros, edges)
    return _mlp2(h, q, W3, b3)


# speedup vs baseline: 3.9345x; 1.3492x over previous
import functools

import jax
import jax.numpy as jnp
from jax import lax
from jax.experimental import pallas as pl
from jax.experimental.pallas import tpu as pltpu
from jax.experimental.pallas import tpu_sc as plsc

N = 10000
E = 320000
D = 128

NC = 2
NS = 16
NW = NC * NS
BLK = 80
WBLK = E // (NW * BLK)
CH = 25
NCHUNK = WBLK // CH


def _sc_aggregate(values, zeros, edges):
    mesh = plsc.VectorSubcoreMesh(core_axis_name="c", subcore_axis_name="s")

    @functools.partial(
        pl.kernel,
        out_type=jax.ShapeDtypeStruct((NC, N, D), jnp.float32),
        mesh=mesh,
        scratch_types=[
            pltpu.VMEM((CH, BLK), jnp.int32),
            pltpu.VMEM((CH, BLK), jnp.int32),
            pltpu.VMEM((BLK, D), jnp.float32),
            pltpu.VMEM((BLK, D), jnp.float32),
            pltpu.VMEM((BLK, D), jnp.float32),
            pltpu.VMEM_SHARED((N, D), jnp.float32),
            pltpu.SemaphoreType.DMA,
            pltpu.SemaphoreType.DMA,
            pltpu.SemaphoreType.DMA,
            pltpu.SemaphoreType.DMA,
            pltpu.SemaphoreType.DMA,
            pltpu.SemaphoreType.DMA,
        ],
    )
    def agg_kernel(x_hbm, z_hbm, e_hbm, out_hbm,
                   src_v, dst_v, rows0, rows1, rows2, acc_sh,
                   g0, g1, g2, s0, s1, s2):
        cid = lax.axis_index("c")
        sid = lax.axis_index("s")
        wid = sid * NC + cid

        @pl.when(sid == 0)
        def _():
            pltpu.sync_copy(z_hbm, acc_sh)

        plsc.subcore_barrier()

        rows = (rows0, rows1, rows2)
        gsem = (g0, g1, g2)
        ssem = (s0, s1, s2)
        gwait = [pltpu.make_async_copy(x_hbm.at[src_v.at[0]], rows[k], gsem[k])
                 for k in range(3)]
        swait = [pltpu.make_async_copy(rows[k], acc_sh.at[dst_v.at[0]],
                                       ssem[k]) for k in range(3)]

        def gath(b, k):
            pltpu.async_copy(x_hbm.at[src_v.at[b]], rows[k], gsem[k])

        def scat(b, k):
            pltpu.async_copy(rows[k], acc_sh.at[dst_v.at[b]], ssem[k],
                             add=True)

        @pl.loop(0, NCHUNK)
        def _(c):
            pltpu.sync_copy(e_hbm.at[0, wid, c], src_v)
            pltpu.sync_copy(e_hbm.at[1, wid, c], dst_v)
            gath(0, 0)
            gath(1, 1)
            gwait[0].wait()
            scat(0, 0)
            gath(2, 2)

            @pl.loop(0, (CH - 4) // 3)
            def _(m):
                j = 3 * m + 1
                gwait[1].wait()
                scat(j, 1)
                swait[0].wait()
                gath(j + 2, 0)
                gwait[2].wait()
                scat(j + 1, 2)
                swait[1].wait()
                gath(j + 3, 1)
                gwait[0].wait()
                scat(j + 2, 0)
                swait[2].wait()
                gath(j + 4, 2)

            gwait[1].wait()
            scat(CH - 3, 1)
            swait[0].wait()
            gath(CH - 1, 0)
            gwait[2].wait()
            scat(CH - 2, 2)
            gwait[0].wait()
            scat(CH - 1, 0)
            swait[1].wait()
            swait[2].wait()
            swait[0].wait()

        plsc.subcore_barrier()

        @pl.when(sid == 0)
        def _():
            pltpu.sync_copy(acc_sh, out_hbm.at[cid])

    return agg_kernel(values, zeros, edges)


ROW_BLK = 1000


def _mlp1_body(x_ref, p_ref, w1_ref, b1_ref, w2_ref, b2_ref, o_ref):
    h = x_ref[...] + p_ref[0] + p_ref[1]
    a = lax.dot_general(h, w1_ref[...], (((1,), (0,)), ((), ())),
                        precision=lax.Precision.HIGHEST,
                        preferred_element_type=jnp.float32)
    a = jnp.maximum(a + b1_ref[...], 0.0)
    hh = lax.dot_general(a, w2_ref[...], (((1,), (0,)), ((), ())),
                         precision=lax.Precision.HIGHEST,
                         preferred_element_type=jnp.float32)
    hh = hh + b2_ref[...]
    o_ref[...] = jnp.where(hh > 0, hh, jnp.exp(hh) - 1.0)


def _mlp2_body(h_ref, q_ref, w3_ref, b3_ref, o_ref):
    h2 = h_ref[...] + q_ref[0] + q_ref[1]
    a = lax.dot_general(h2, w3_ref[...], (((1,), (0,)), ((), ())),
                        precision=lax.Precision.HIGHEST,
                        preferred_element_type=jnp.float32)
    o_ref[...] = jnp.maximum(a + b3_ref[...], 0.0)


def _row_spec():
    return pl.BlockSpec((ROW_BLK, D), lambda i: (i, 0))


def _pair_spec():
    return pl.BlockSpec((NC, ROW_BLK, D), lambda i: (0, i, 0))


def _full_spec(shape):
    return pl.BlockSpec(shape, lambda i: tuple(0 for _ in shape))


def _mlp1(x, p, W1, b1, W2, b2):
    return pl.pallas_call(
        _mlp1_body,
        grid=(N // ROW_BLK,),
        in_specs=[_row_spec(), _pair_spec(),
                  _full_spec((D, D)), _full_spec((1, D)),
                  _full_spec((D, D)), _full_spec((1, D))],
        out_specs=_row_spec(),
        out_shape=jax.ShapeDtypeStruct((N, D), jnp.float32),
        compiler_params=pltpu.CompilerParams(
            dimension_semantics=("parallel",)),
    )(x, p, W1, b1.reshape(1, D), W2, b2.reshape(1, D))


def _mlp2(h, q, W3, b3):
    return pl.pallas_call(
        _mlp2_body,
        grid=(N // ROW_BLK,),
        in_specs=[_row_spec(), _pair_spec(),
                  _full_spec((D, D)), _full_spec((1, D))],
        out_specs=_row_spec(),
        out_shape=jax.ShapeDtypeStruct((N, D), jnp.float32),
        compiler_params=pltpu.CompilerParams(
            dimension_semantics=("parallel",)),
    )(h, q, W3, b3.reshape(1, D))


def kernel(x, edge_index, W1, b1, W2, b2, W3, b3):
    edges = edge_index.astype(jnp.int32).reshape(2, NW, NCHUNK, CH, BLK)
    zeros = jnp.zeros((N, D), jnp.float32)

    p = _sc_aggregate(x, zeros, edges)
    h = _mlp1(x, p, W1, b1, W2, b2)
    q = _sc_aggregate(h, zeros, edges)
    return _mlp2(h, q, W3, b3)


# speedup vs baseline: 4.3021x; 1.0934x over previous
import functools

import jax
import jax.numpy as jnp
from jax import lax
from jax.experimental import pallas as pl
from jax.experimental.pallas import tpu as pltpu
from jax.experimental.pallas import tpu_sc as plsc

N = 10000
E = 320000
D = 128

NC = 2
NS = 16
NW = NC * NS
BLK = 80
WBLK = E // (NW * BLK)
CH = 25
NCHUNK = WBLK // CH


def _sc_aggregate(values, zeros, edges):
    mesh = plsc.VectorSubcoreMesh(core_axis_name="c", subcore_axis_name="s")

    @functools.partial(
        pl.kernel,
        out_type=jax.ShapeDtypeStruct((NC, N, D), jnp.float32),
        mesh=mesh,
        scratch_types=[
            pltpu.VMEM((CH, BLK), jnp.int32),
            pltpu.VMEM((CH, BLK), jnp.int32),
            pltpu.VMEM((BLK, D), jnp.float32),
            pltpu.VMEM((BLK, D), jnp.float32),
            pltpu.VMEM((BLK, D), jnp.float32),
            pltpu.VMEM_SHARED((N, D), jnp.float32),
            pltpu.SemaphoreType.DMA,
            pltpu.SemaphoreType.DMA,
            pltpu.SemaphoreType.DMA,
            pltpu.SemaphoreType.DMA,
            pltpu.SemaphoreType.DMA,
            pltpu.SemaphoreType.DMA,
        ],
    )
    def agg_kernel(x_hbm, z_hbm, e_hbm, out_hbm,
                   src_v, dst_v, rows0, rows1, rows2, acc_sh,
                   g0, g1, g2, s0, s1, s2):
        cid = lax.axis_index("c")
        sid = lax.axis_index("s")
        wid = sid * NC + cid

        off = pl.multiple_of(sid * 624, 8)

        @pl.when(sid < 15)
        def _():
            pltpu.sync_copy(z_hbm.at[pl.ds(off, 624)],
                            acc_sh.at[pl.ds(off, 624)])

        @pl.when(sid == 15)
        def _():
            pltpu.sync_copy(z_hbm.at[pl.ds(9360, 640)],
                            acc_sh.at[pl.ds(9360, 640)])

        plsc.subcore_barrier()

        rows = (rows0, rows1, rows2)
        gsem = (g0, g1, g2)
        ssem = (s0, s1, s2)
        gwait = [pltpu.make_async_copy(x_hbm.at[src_v.at[0]], rows[k], gsem[k])
                 for k in range(3)]
        swait = [pltpu.make_async_copy(rows[k], acc_sh.at[dst_v.at[0]],
                                       ssem[k]) for k in range(3)]

        def gath(b, k):
            pltpu.async_copy(x_hbm.at[src_v.at[b]], rows[k], gsem[k])

        def scat(b, k):
            pltpu.async_copy(rows[k], acc_sh.at[dst_v.at[b]], ssem[k],
                             add=True)

        @pl.loop(0, NCHUNK)
        def _(c):
            pltpu.sync_copy(e_hbm.at[0, wid, c], src_v)
            pltpu.sync_copy(e_hbm.at[1, wid, c], dst_v)
            gath(0, 0)
            gath(1, 1)
            gwait[0].wait()
            scat(0, 0)
            gath(2, 2)

            @pl.loop(0, (CH - 4) // 3)
            def _(m):
                j = 3 * m + 1
                gwait[1].wait()
                scat(j, 1)
                swait[0].wait()
                gath(j + 2, 0)
                gwait[2].wait()
                scat(j + 1, 2)
                swait[1].wait()
                gath(j + 3, 1)
                gwait[0].wait()
                scat(j + 2, 0)
                swait[2].wait()
                gath(j + 4, 2)

            gwait[1].wait()
            scat(CH - 3, 1)
            swait[0].wait()
            gath(CH - 1, 0)
            gwait[2].wait()
            scat(CH - 2, 2)
            gwait[0].wait()
            scat(CH - 1, 0)
            swait[1].wait()
            swait[2].wait()
            swait[0].wait()

        plsc.subcore_barrier()

        @pl.when(sid < 15)
        def _():
            pltpu.sync_copy(acc_sh.at[pl.ds(off, 624)],
                            out_hbm.at[cid, pl.ds(off, 624)])

        @pl.when(sid == 15)
        def _():
            pltpu.sync_copy(acc_sh.at[pl.ds(9360, 640)],
                            out_hbm.at[cid, pl.ds(9360, 640)])

    return agg_kernel(values, zeros, edges)


ROW_BLK = 1000


def _mlp1_body(x_ref, p_ref, w1_ref, b1_ref, w2_ref, b2_ref, o_ref):
    h = x_ref[...] + p_ref[0] + p_ref[1]
    a = lax.dot_general(h, w1_ref[...], (((1,), (0,)), ((), ())),
                        precision=lax.Precision.DEFAULT,
                        preferred_element_type=jnp.float32)
    a = jnp.maximum(a + b1_ref[...], 0.0)
    hh = lax.dot_general(a, w2_ref[...], (((1,), (0,)), ((), ())),
                         precision=lax.Precision.DEFAULT,
                         preferred_element_type=jnp.float32)
    hh = hh + b2_ref[...]
    o_ref[...] = jnp.where(hh > 0, hh, jnp.exp(hh) - 1.0)


def _mlp2_body(h_ref, q_ref, w3_ref, b3_ref, o_ref):
    h2 = h_ref[...] + q_ref[0] + q_ref[1]
    a = lax.dot_general(h2, w3_ref[...], (((1,), (0,)), ((), ())),
                        precision=lax.Precision.DEFAULT,
                        preferred_element_type=jnp.float32)
    o_ref[...] = jnp.maximum(a + b3_ref[...], 0.0)


def _row_spec():
    return pl.BlockSpec((ROW_BLK, D), lambda i: (i, 0))


def _pair_spec():
    return pl.BlockSpec((NC, ROW_BLK, D), lambda i: (0, i, 0))


def _full_spec(shape):
    return pl.BlockSpec(shape, lambda i: tuple(0 for _ in shape))


def _mlp1(x, p, W1, b1, W2, b2):
    return pl.pallas_call(
        _mlp1_body,
        grid=(N // ROW_BLK,),
        in_specs=[_row_spec(), _pair_spec(),
                  _full_spec((D, D)), _full_spec((1, D)),
                  _full_spec((D, D)), _full_spec((1, D))],
        out_specs=_row_spec(),
        out_shape=jax.ShapeDtypeStruct((N, D), jnp.float32),
        compiler_params=pltpu.CompilerParams(
            dimension_semantics=("parallel",)),
    )(x, p, W1, b1.reshape(1, D), W2, b2.reshape(1, D))


def _mlp2(h, q, W3, b3):
    return pl.pallas_call(
        _mlp2_body,
        grid=(N // ROW_BLK,),
        in_specs=[_row_spec(), _pair_spec(),
                  _full_spec((D, D)), _full_spec((1, D))],
        out_specs=_row_spec(),
        out_shape=jax.ShapeDtypeStruct((N, D), jnp.float32),
        compiler_params=pltpu.CompilerParams(
            dimension_semantics=("parallel",)),
    )(h, q, W3, b3.reshape(1, D))


def kernel(x, edge_index, W1, b1, W2, b2, W3, b3):
    edges = edge_index.astype(jnp.int32).reshape(2, NW, NCHUNK, CH, BLK)
    zeros = jnp.zeros((N, D), jnp.float32)

    p = _sc_aggregate(x, zeros, edges)
    h = _mlp1(x, p, W1, b1, W2, b2)
    q = _sc_aggregate(h, zeros, edges)
    return _mlp2(h, q, W3, b3)
